# baseline (speedup 1.0000x reference)
import math

import jax
import jax.numpy as jnp
from jax import lax
from jax.experimental import pallas as pl
from jax.experimental.pallas import tpu as pltpu
from jax.experimental.pallas import tpu_sc as plsc

VOCAB_SIZE = 1000000
D = 64
SCALE = math.sqrt(D)

NC = 2
NS = 16
NW = NC * NS
L = 16
BB = 128
T = 200


def _emb_body(x_hbm, lut_hbm, out_hbm, idx_v, xT_v, rows, rowsT, gsem, ssem):
  wid = lax.axis_index("s") * NC + lax.axis_index("c")
  b0 = wid * BB
  pltpu.sync_copy(x_hbm.at[pl.ds(b0, BB)], idx_v)

  iota = lax.iota(jnp.int32, L)

  @plsc.parallel_loop(0, T * (BB // L), unroll=8)
  def _(q):
    t = q >> 3
    l0 = (q & 7) * L
    vals = plsc.load_gather(idx_v, [l0 + iota, jnp.zeros((L,), jnp.int32) + t])
    xT_v[t, pl.ds(l0, L)] = vals

  def start_gather(t, b):
    pltpu.make_async_copy(
        lut_hbm.at[xT_v.at[t]], rows.at[b], gsem.at[b]
    ).start()

  def wait_gather(b):
    pltpu.make_async_copy(
        lut_hbm.at[xT_v.at[0]], rows.at[b], gsem.at[b]
    ).wait()

  def start_store(t, b):
    pltpu.make_async_copy(
        rowsT.at[b], out_hbm.at[t, :, wid], ssem.at[b]
    ).start()

  def wait_store(b):
    pltpu.make_async_copy(
        rowsT.at[b], out_hbm.at[0, :, 0], ssem.at[b]
    ).wait()

  def transpose_scale(b):
    @plsc.parallel_loop(0, D * (BB // L), unroll=8)
    def _(q):
      f = q >> 3
      l0 = (q & 7) * L
      vals = plsc.load_gather(
          rows.at[b], [l0 + iota, jnp.zeros((L,), jnp.int32) + f])
      rowsT[b, f >> 3, f & 7, pl.ds(l0, L)] = vals * SCALE

  def step(t, b, first, last):
    wait_gather(b)
    if not first:
      wait_store(b)
    transpose_scale(b)
    start_store(t, b)
    if not last:
      start_gather(t + 2, b)

  start_gather(0, 0)
  start_gather(1, 1)
  step(0, 0, True, False)
  step(1, 1, True, False)

  def loop_body(tt, _):
    t0 = tt * 2
    step(t0, 0, False, False)
    step(t0 + 1, 1, False, False)
    return 0

  lax.fori_loop(1, T // 2 - 1, loop_body, 0)
  t0 = T - 2
  step(t0, 0, False, True)
  step(t0 + 1, 1, False, True)
  wait_store(0)
  wait_store(1)


@jax.jit
def kernel(x, lut):
  B = x.shape[0]
  mesh = plsc.VectorSubcoreMesh(core_axis_name="c", subcore_axis_name="s")
  out5 = pl.kernel(
      _emb_body,
      out_type=jax.ShapeDtypeStruct((T, 8, B // BB, 8, BB), jnp.float32),
      mesh=mesh,
      scratch_types=[
          pltpu.VMEM((BB, T), jnp.int32),
          pltpu.VMEM((T, BB), jnp.int32),
          pltpu.VMEM((2, BB, D), jnp.float32),
          pltpu.VMEM((2, 8, 8, BB), jnp.float32),
          pltpu.SemaphoreType.DMA((2,)),
          pltpu.SemaphoreType.DMA((2,)),
      ],
      compiler_params=pltpu.CompilerParams(
          use_tc_tiling_on_sc=False, needs_layout_passes=False),
  )(x, lut)
  return out5.transpose(2, 4, 0, 1, 3).reshape(B, T, D)


# speedup vs baseline: 1.1995x; 1.1995x over previous
import math

import jax
import jax.numpy as jnp
from jax import lax
from jax.experimental import pallas as pl
from jax.experimental.pallas import tpu as pltpu
from jax.experimental.pallas import tpu_sc as plsc

VOCAB_SIZE = 1000000
D = 64
SCALE = math.sqrt(D)

NC = 2
NS = 16
NW = NC * NS
L = 16
BB = 128
T = 200
K = 2
NSTEP = T // K


def _emb_body(x_hbm, lut_hbm, out_hbm, idx_v, xT_v, rows, rowsT, gsem, ssem):
  wid = lax.axis_index("s") * NC + lax.axis_index("c")
  b0 = wid * BB
  pltpu.sync_copy(x_hbm.at[pl.ds(b0, BB)], idx_v)

  iota = lax.iota(jnp.int32, L)

  @plsc.parallel_loop(0, T * (BB // L), unroll=8)
  def _(q):
    t = q >> 3
    l0 = (q & 7) * L
    vals = plsc.load_gather(idx_v, [l0 + iota, jnp.zeros((L,), jnp.int32) + t])
    xT_v[pl.ds(t * BB + l0, L)] = vals

  def start_gather(s, b):
    pltpu.make_async_copy(
        lut_hbm.at[xT_v.at[pl.ds(s * (K * BB), K * BB)]], rows.at[b],
        gsem.at[b]).start()

  def wait_gather(b):
    pltpu.make_async_copy(
        lut_hbm.at[xT_v.at[pl.ds(0, K * BB)]], rows.at[b], gsem.at[b]
    ).wait()

  def start_store(s, b):
    pltpu.make_async_copy(
        rowsT.at[b], out_hbm.at[pl.ds(s * K, K), :, wid], ssem.at[b]
    ).start()

  def wait_store(b):
    pltpu.make_async_copy(
        rowsT.at[b], out_hbm.at[pl.ds(0, K), :, 0], ssem.at[b]
    ).wait()

  def transpose_scale(b):
    @plsc.parallel_loop(0, K * D, unroll=2)
    def _(q):
      k = q >> 6
      f = q & (D - 1)
      cvec = jnp.zeros((L,), jnp.int32) + f
      rbase = k * BB
      for g in range(BB // L):
        vals = plsc.load_gather(
            rows.at[b], [rbase + g * L + iota, cvec])
        rowsT[b, k, f >> 3, f & 7, pl.ds(g * L, L)] = vals * SCALE

  def step(s, b, first, last):
    wait_gather(b)
    if not first:
      wait_store(b)
    transpose_scale(b)
    start_store(s, b)
    if not last:
      start_gather(s + 2, b)

  start_gather(0, 0)
  start_gather(1, 1)
  step(0, 0, True, False)
  step(1, 1, True, False)

  def loop_body(ss, _):
    s0 = ss * 2
    step(s0, 0, False, False)
    step(s0 + 1, 1, False, False)
    return 0

  lax.fori_loop(1, NSTEP // 2 - 1, loop_body, 0)
  s0 = NSTEP - 2
  step(s0, 0, False, True)
  step(s0 + 1, 1, False, True)
  wait_store(0)
  wait_store(1)


@jax.jit
def kernel(x, lut):
  B = x.shape[0]
  mesh = plsc.VectorSubcoreMesh(core_axis_name="c", subcore_axis_name="s")
  out5 = pl.kernel(
      _emb_body,
      out_type=jax.ShapeDtypeStruct((T, 8, B // BB, 8, BB), jnp.float32),
      mesh=mesh,
      scratch_types=[
          pltpu.VMEM((BB, T), jnp.int32),
          pltpu.VMEM((T * BB,), jnp.int32),
          pltpu.VMEM((2, K * BB, D), jnp.float32),
          pltpu.VMEM((2, K, 8, 8, BB), jnp.float32),
          pltpu.SemaphoreType.DMA((2,)),
          pltpu.SemaphoreType.DMA((2,)),
      ],
      compiler_params=pltpu.CompilerParams(
          use_tc_tiling_on_sc=False, needs_layout_passes=False),
  )(x, lut)
  return out5.transpose(2, 4, 0, 1, 3).reshape(B, T, D)


# speedup vs baseline: 1.2089x; 1.0078x over previous
import math

import jax
import jax.numpy as jnp
from jax import lax
from jax.experimental import pallas as pl
from jax.experimental.pallas import tpu as pltpu
from jax.experimental.pallas import tpu_sc as plsc

VOCAB_SIZE = 1000000
D = 64
SCALE = math.sqrt(D)

NC = 2
NS = 16
NW = NC * NS
L = 16
BB = 128
T = 200
K = 2
NSTEP = T // K


def _emb_body(x_hbm, lut_hbm, out_hbm, idx_v, xT_v, rows, rowsT, gsem, ssem):
  wid = lax.axis_index("s") * NC + lax.axis_index("c")
  b0 = wid * BB
  pltpu.sync_copy(x_hbm.at[pl.ds(b0, BB)], idx_v)

  iota = lax.iota(jnp.int32, L)

  @plsc.parallel_loop(0, T * (BB // L), unroll=8)
  def _(q):
    t = q >> 3
    l0 = (q & 7) * L
    vals = plsc.load_gather(idx_v, [l0 + iota, jnp.zeros((L,), jnp.int32) + t])
    xT_v[pl.ds(t * BB + l0, L)] = vals

  def start_gather(s, b):
    pltpu.make_async_copy(
        lut_hbm.at[xT_v.at[pl.ds(s * (K * BB), K * BB)]], rows.at[b],
        gsem.at[b]).start()

  def wait_gather(b):
    pltpu.make_async_copy(
        lut_hbm.at[xT_v.at[pl.ds(0, K * BB)]], rows.at[b], gsem.at[b]
    ).wait()

  def start_store(s, b):
    pltpu.make_async_copy(
        rowsT.at[b], out_hbm.at[pl.ds(s * K, K), :, wid], ssem.at[b]
    ).start()

  def wait_store(b):
    pltpu.make_async_copy(
        rowsT.at[b], out_hbm.at[pl.ds(0, K), :, 0], ssem.at[b]
    ).wait()

  def transpose_scale(b):
    @plsc.parallel_loop(0, K * D, unroll=8)
    def _(q):
      k = q >> 6
      f = q & (D - 1)
      cvec = jnp.zeros((L,), jnp.int32) + f
      rbase = k * BB
      for g in range(BB // L):
        vals = plsc.load_gather(
            rows.at[b], [rbase + g * L + iota, cvec])
        rowsT[b, k, f >> 3, f & 7, pl.ds(g * L, L)] = vals * SCALE

  def step(s, b, first, last):
    wait_gather(b)
    if not first:
      wait_store(b)
    transpose_scale(b)
    start_store(s, b)
    if not last:
      start_gather(s + 2, b)

  start_gather(0, 0)
  start_gather(1, 1)
  step(0, 0, True, False)
  step(1, 1, True, False)

  def loop_body(ss, _):
    s0 = ss * 2
    step(s0, 0, False, False)
    step(s0 + 1, 1, False, False)
    return 0

  lax.fori_loop(1, NSTEP // 2 - 1, loop_body, 0)
  s0 = NSTEP - 2
  step(s0, 0, False, True)
  step(s0 + 1, 1, False, True)
  wait_store(0)
  wait_store(1)


@jax.jit
def kernel(x, lut):
  B = x.shape[0]
  mesh = plsc.VectorSubcoreMesh(core_axis_name="c", subcore_axis_name="s")
  out5 = pl.kernel(
      _emb_body,
      out_type=jax.ShapeDtypeStruct((T, 8, B // BB, 8, BB), jnp.float32),
      mesh=mesh,
      scratch_types=[
          pltpu.VMEM((BB, T), jnp.int32),
          pltpu.VMEM((T * BB,), jnp.int32),
          pltpu.VMEM((2, K * BB, D), jnp.float32),
          pltpu.VMEM((2, K, 8, 8, BB), jnp.float32),
          pltpu.SemaphoreType.DMA((2,)),
          pltpu.SemaphoreType.DMA((2,)),
      ],
      compiler_params=pltpu.CompilerParams(
          use_tc_tiling_on_sc=False, needs_layout_passes=False),
  )(x, lut)
  return out5.transpose(2, 4, 0, 1, 3).reshape(B, T, D)


# speedup vs baseline: 1.6990x; 1.4055x over previous
import math

import jax
import jax.numpy as jnp
from jax import lax
from jax.experimental import pallas as pl
from jax.experimental.pallas import tpu as pltpu
from jax.experimental.pallas import tpu_sc as plsc

VOCAB_SIZE = 1000000
D = 64
SCALE = math.sqrt(D)

NC = 2
NS = 16
NW = NC * NS
L = 16
BB = 128
T = 200
K = 2
NSTEP = T // K


def _emb_body(x_hbm, lut_hbm, out_hbm, idx_v, xT_v, rows, rowsT, gsem, ssem):
  wid = lax.axis_index("s") * NC + lax.axis_index("c")
  b0 = wid * BB
  pltpu.sync_copy(x_hbm.at[pl.ds(b0, BB)], idx_v)

  iota = lax.iota(jnp.int32, L)

  @plsc.parallel_loop(0, T * (BB // L), unroll=8)
  def _(q):
    t = q >> 3
    l0 = (q & 7) * L
    vals = plsc.load_gather(idx_v, [l0 + iota, jnp.zeros((L,), jnp.int32) + t])
    xT_v[pl.ds(t * BB + l0, L)] = vals

  def start_gather(s, b):
    pltpu.make_async_copy(
        lut_hbm.at[xT_v.at[pl.ds(s * (K * BB), K * BB)]],
        rows.at[b], gsem.at[b]).start()

  def wait_gather(b):
    pltpu.make_async_copy(
        lut_hbm.at[xT_v.at[pl.ds(0, K * BB)]], rows.at[b],
        gsem.at[b]).wait()

  def start_store(s, b):
    pltpu.make_async_copy(
        rowsT.at[b], out_hbm.at[pl.ds(s * K, K), :, wid], ssem.at[b]
    ).start()

  def wait_store(b):
    pltpu.make_async_copy(
        rowsT.at[b], out_hbm.at[pl.ds(0, K), :, 0], ssem.at[b]
    ).wait()

  def transpose_scale(b):
    @plsc.parallel_loop(0, K * (D // L) * (BB // L), unroll=2)
    def _(q):
      k = q >> 5
      f0 = ((q >> 3) & 3) * L
      l0 = (q & 7) * L
      row0 = k * BB + l0
      for d in range(L):
        fcol = f0 + ((d + iota) & (L - 1))
        vals = plsc.load_gather(rows.at[b], [row0 + iota, fcol])
        plsc.store_scatter(
            rowsT.at[b, k],
            [fcol >> 3, fcol & 7, l0 + iota],
            vals * SCALE)

  def step(s, b, first, last):
    wait_gather(b)
    if not first:
      wait_store(b)
    transpose_scale(b)
    start_store(s, b)
    if not last:
      start_gather(s + 2, b)

  start_gather(0, 0)
  start_gather(1, 1)
  step(0, 0, True, False)
  step(1, 1, True, False)

  def loop_body(ss, _):
    s0 = ss * 2
    step(s0, 0, False, False)
    step(s0 + 1, 1, False, False)
    return 0

  lax.fori_loop(1, NSTEP // 2 - 1, loop_body, 0)
  s0 = NSTEP - 2
  step(s0, 0, False, True)
  step(s0 + 1, 1, False, True)
  wait_store(0)
  wait_store(1)


@jax.jit
def kernel(x, lut):
  B = x.shape[0]
  mesh = plsc.VectorSubcoreMesh(core_axis_name="c", subcore_axis_name="s")
  out5 = pl.kernel(
      _emb_body,
      out_type=jax.ShapeDtypeStruct((T, 8, B // BB, 8, BB), jnp.float32),
      mesh=mesh,
      scratch_types=[
          pltpu.VMEM((BB, T), jnp.int32),
          pltpu.VMEM((T * BB,), jnp.int32),
          pltpu.VMEM((2, K * BB, D), jnp.float32),
          pltpu.VMEM((2, K, 8, 8, BB), jnp.float32),
          pltpu.SemaphoreType.DMA((2,)),
          pltpu.SemaphoreType.DMA((2,)),
      ],
      compiler_params=pltpu.CompilerParams(
          use_tc_tiling_on_sc=False, needs_layout_passes=False),
  )(x, lut)
  return out5.transpose(2, 4, 0, 1, 3).reshape(B, T, D)


# speedup vs baseline: 2.9521x; 1.7376x over previous
import math

import jax
import jax.numpy as jnp
from jax import lax
from jax.experimental import pallas as pl
from jax.experimental.pallas import tpu as pltpu
from jax.experimental.pallas import tpu_sc as plsc

VOCAB_SIZE = 1000000
D = 64
SCALE = math.sqrt(D)

NC = 2
NS = 16
NW = NC * NS
L = 16
BB = 128
T = 200
K = 2
NSTEP = T // K
XCH = 16

PACK_BLK = 8192


def _pack_body(lutT_ref, out_ref):
  a = lutT_ref[...] * SCALE
  out_ref[:, 0:D] = a.T


def _pack_table(lutT):
  grid = (VOCAB_SIZE + PACK_BLK - 1) // PACK_BLK
  return pl.pallas_call(
      _pack_body,
      grid=(grid,),
      in_specs=[pl.BlockSpec((D, PACK_BLK), lambda i: (0, i))],
      out_specs=pl.BlockSpec((PACK_BLK, 2 * D), lambda i: (i, 0)),
      out_shape=jax.ShapeDtypeStruct((VOCAB_SIZE, 2 * D), jnp.float32),
  )(lutT)


def _emb_body(x_hbm, tab_hbm, out_hbm, xbuf, xT, rows, rowsT,
              gsem, ssem):
  wid = lax.axis_index("s") * NC + lax.axis_index("c")
  b0 = wid * BB

  iota = lax.iota(jnp.int32, L)

  for c in range(BB // XCH):
    pltpu.sync_copy(x_hbm.at[pl.ds(b0 + c * XCH, XCH)], xbuf)

    @plsc.parallel_loop(0, T, unroll=4)
    def _(t):
      vals = plsc.load_gather(xbuf, [iota, jnp.zeros((L,), jnp.int32) + t])
      xT[pl.ds(t * BB + c * XCH, L)] = vals

  def start_gather(s, b):
    pltpu.make_async_copy(
        tab_hbm.at[xT.at[pl.ds(s * (K * BB), K * BB)]], rows.at[b],
        gsem.at[b]).start()

  def wait_gather(b):
    pltpu.make_async_copy(
        tab_hbm.at[xT.at[pl.ds(0, K * BB)]], rows.at[b], gsem.at[b]).wait()

  def start_store(s, b):
    pltpu.make_async_copy(
        rowsT.at[b], out_hbm.at[pl.ds(s * K, K), :, wid], ssem.at[b]
    ).start()

  def wait_store(b):
    pltpu.make_async_copy(
        rowsT.at[b], out_hbm.at[pl.ds(0, K), :, 0], ssem.at[b]
    ).wait()

  def transpose_scale(b):
    @plsc.parallel_loop(0, K * (D // L) * (BB // L), unroll=2)
    def _(q):
      k = q >> 5
      f0 = ((q >> 3) & 3) * L
      l0 = (q & 7) * L
      row0 = k * BB + l0
      for d in range(L):
        fd = (d + iota) & (L - 1)
        vals = plsc.load_gather(rows.at[b], [row0 + iota, f0 + fd])
        plsc.store_scatter(
            rowsT.at[b, k],
            [(f0 + fd) >> 3, (f0 + fd) & 7, l0 + iota],
            vals)

  def step(s, b, first, last):
    wait_gather(b)
    if not first:
      wait_store(b)
    transpose_scale(b)
    start_store(s, b)
    if not last:
      start_gather(s + 2, b)

  start_gather(0, 0)
  start_gather(1, 1)

  step(0, 0, True, False)
  step(1, 1, True, False)

  def loop_body(ss, _):
    s0 = ss * 2
    step(s0, 0, False, False)
    step(s0 + 1, 1, False, False)
    return 0

  lax.fori_loop(1, NSTEP // 2 - 1, loop_body, 0)
  step(NSTEP - 2, 0, False, True)
  step(NSTEP - 1, 1, False, True)
  wait_store(0)
  wait_store(1)


@jax.jit
def kernel(x, lut):
  B = x.shape[0]
  tab = _pack_table(jnp.swapaxes(lut, 0, 1))
  mesh = plsc.VectorSubcoreMesh(core_axis_name="c", subcore_axis_name="s")
  out5 = pl.kernel(
      _emb_body,
      out_type=jax.ShapeDtypeStruct((T, 8, B // BB, 8, BB), jnp.float32),
      mesh=mesh,
      scratch_types=[
          pltpu.VMEM((XCH, T), jnp.int32),
          pltpu.VMEM((T * BB,), jnp.int32),
          pltpu.VMEM((2, K * BB, 2 * D), jnp.float32),
          pltpu.VMEM((2, K, 8, 8, BB), jnp.float32),
          pltpu.SemaphoreType.DMA((2,)),
          pltpu.SemaphoreType.DMA((2,)),
      ],
      compiler_params=pltpu.CompilerParams(
          use_tc_tiling_on_sc=False, needs_layout_passes=False),
  )(x, tab)
  return out5.transpose(2, 4, 0, 1, 3).reshape(B, T, D)
